# Initial kernel scaffold; baseline (speedup 1.0000x reference)
import jax
import jax.numpy as jnp
from jax.experimental import pallas as pl


def kernel(x_data, x_tasks, x_devices, counts, edge_index_dt, edge_index_tt, Wd_src, Wd_dst, ad_src, ad_dst, W1, a1_src, a1_dst, W2, a2_src, a2_dst, W_dev):
    raise NotImplementedError("write your pallas kernel here")


# speedup vs baseline: 1.7740x; 1.7740x over previous
import functools
import jax
import jax.numpy as jnp
from jax.experimental import pallas as pl

H = 128
N = 10000
E = 320000


def _prologue_body(xd_ref, xt_ref, ws_ref, wd_ref, asrc_ref, adst_ref,
                   h_ref, als_ref, ald_ref):
    h_src = jnp.dot(xd_ref[...], ws_ref[...], preferred_element_type=jnp.float32)
    h_ref[...] = h_src
    als_ref[...] = jnp.dot(h_src, asrc_ref[...], preferred_element_type=jnp.float32)
    h_dst = jnp.dot(xt_ref[...], wd_ref[...], preferred_element_type=jnp.float32)
    ald_ref[...] = jnp.dot(h_dst, adst_ref[...], preferred_element_type=jnp.float32)


def _prologue(x_data, x_tasks, Wd_src, Wd_dst, ad_src, ad_dst):
    return pl.pallas_call(
        _prologue_body,
        out_shape=(
            jax.ShapeDtypeStruct((N, H), jnp.float32),
            jax.ShapeDtypeStruct((N,), jnp.float32),
            jax.ShapeDtypeStruct((N,), jnp.float32),
        ),
    )(x_data, x_tasks, Wd_src, Wd_dst, ad_src, ad_dst)


def _mid_body(acc_ref, den_ref, w_ref, asrc_ref, adst_ref,
              h_ref, als_ref, ald_ref):
    t = acc_ref[...] / (den_ref[...] + 1e-16)[:, None]
    t = jnp.where(t > 0, t, jnp.exp(t) - 1.0)
    h = jnp.dot(t, w_ref[...], preferred_element_type=jnp.float32)
    h_ref[...] = h
    als_ref[...] = jnp.dot(h, asrc_ref[...], preferred_element_type=jnp.float32)
    ald_ref[...] = jnp.dot(h, adst_ref[...], preferred_element_type=jnp.float32)


def _mid(acc, den, W, a_src, a_dst):
    return pl.pallas_call(
        _mid_body,
        out_shape=(
            jax.ShapeDtypeStruct((N, H), jnp.float32),
            jax.ShapeDtypeStruct((N,), jnp.float32),
            jax.ShapeDtypeStruct((N,), jnp.float32),
        ),
    )(acc, den, W, a_src, a_dst)


def _epilogue_body(acc_ref, den_ref, xdev_ref, wdev_ref, counts_ref, out_ref):
    t = acc_ref[...] / (den_ref[...] + 1e-16)[:, None]
    t = jnp.where(t > 0, t, jnp.exp(t) - 1.0)
    out_ref[0:H] = t[0]
    cnt = jnp.maximum(counts_ref[0, 0], 1.0)
    out_ref[H:2 * H] = jnp.sum(t, axis=0) / cnt
    dev = jnp.maximum(
        jnp.dot(xdev_ref[...], wdev_ref[...], preferred_element_type=jnp.float32), 0.0)
    out_ref[2 * H:] = dev.reshape(-1)


def _epilogue(acc, den, x_devices, W_dev, counts):
    ndev = x_devices.shape[0]
    return pl.pallas_call(
        _epilogue_body,
        out_shape=jax.ShapeDtypeStruct((2 * H + ndev * H,), jnp.float32),
    )(acc, den, x_devices, W_dev, counts)


def _edge_phase(h_src, als, ald, src, dst):
    e = als[src] + ald[dst]
    e = jnp.maximum(e, 0.2 * e)
    ex = jnp.exp(e)
    den = jax.ops.segment_sum(ex, dst, num_segments=N)
    acc = jax.ops.segment_sum(h_src[src] * ex[:, None], dst, num_segments=N)
    return acc, den


def kernel(x_data, x_tasks, x_devices, counts, edge_index_dt, edge_index_tt,
           Wd_src, Wd_dst, ad_src, ad_dst,
           W1, a1_src, a1_dst,
           W2, a2_src, a2_dst,
           W_dev):
    src_dt = edge_index_dt[0].astype(jnp.int32)
    dst_dt = edge_index_dt[1].astype(jnp.int32)
    src_tt = edge_index_tt[0].astype(jnp.int32)
    dst_tt = edge_index_tt[1].astype(jnp.int32)

    h, als, ald = _prologue(x_data, x_tasks, Wd_src, Wd_dst, ad_src, ad_dst)
    acc, den = _edge_phase(h, als, ald, src_dt, dst_dt)
    h, als, ald = _mid(acc, den, W1, a1_src, a1_dst)
    acc, den = _edge_phase(h, als, ald, src_tt, dst_tt)
    h, als, ald = _mid(acc, den, W2, a2_src, a2_dst)
    acc, den = _edge_phase(h, als, ald, src_tt, dst_tt)
    return _epilogue(acc, den, x_devices, W_dev, counts)


# speedup vs baseline: 20.5641x; 11.5919x over previous
import functools
import jax
import jax.numpy as jnp
from jax import lax
from jax.experimental import pallas as pl
from jax.experimental.pallas import tpu as pltpu
from jax.experimental.pallas import tpu_sc as plsc

H = 128
N = 10000
E = 320000

NC = 2
NS = 16
NW = NC * NS
L = 16

C = 80
EPW = E // NW
NCH = EPW // C
RPT = 624
RPT_LAST = N - (NS - 1) * RPT


def _edge_body(h_hbm, als_hbm, ald_hbm, src_hbm, dst_hbm,
               acc_hbm, den_hbm,
               als_l, ald_l, src_c, dst_c, ex_v, exs, rows,
               acc_s, den_s):
    cid = lax.axis_index("c")
    sid = lax.axis_index("s")
    wid = sid * NC + cid
    zeros16 = jnp.zeros((L,), jnp.float32)

    def _zero_row(i, carry):
        for c in range(H // L):
            rows[i, pl.ds(c * L, L)] = zeros16
        exs[i, :] = zeros16
        return carry
    lax.fori_loop(0, C, _zero_row, 0)

    row0 = sid * RPT

    def _zero_rows(nrows):
        full = nrows // C
        for k in range(full):
            pltpu.sync_copy(rows.at[pl.ds(0, C)],
                            acc_s.at[pl.ds(row0 + C * k, C)])
            pltpu.sync_copy(exs.at[pl.ds(0, C)],
                            den_s.at[pl.ds(row0 + C * k, C)])
        rem = nrows - full * C
        if rem:
            pltpu.sync_copy(rows.at[pl.ds(0, rem)],
                            acc_s.at[pl.ds(row0 + full * C, rem)])
            pltpu.sync_copy(exs.at[pl.ds(0, rem)],
                            den_s.at[pl.ds(row0 + full * C, rem)])

    pl.when(sid < NS - 1)(lambda: _zero_rows(RPT))
    pl.when(sid == NS - 1)(lambda: _zero_rows(RPT_LAST))

    pltpu.sync_copy(als_hbm, als_l)
    pltpu.sync_copy(ald_hbm, ald_l)

    plsc.subcore_barrier()

    def _chunk(g, carry):
        base = wid * EPW + g * C
        pltpu.sync_copy(src_hbm.at[pl.ds(base, C)], src_c)
        pltpu.sync_copy(dst_hbm.at[pl.ds(base, C)], dst_c)
        for gg in range(C // L):
            sidx = src_c[pl.ds(gg * L, L)]
            didx = dst_c[pl.ds(gg * L, L)]
            s = plsc.load_gather(als_l, [sidx]) + plsc.load_gather(ald_l, [didx])
            e = jnp.maximum(s, 0.2 * s)
            ex_v[pl.ds(gg * L, L)] = jnp.exp(e)
        pltpu.sync_copy(h_hbm.at[src_c], rows)
        def _scale(r, carry2):
            spl = plsc.load_gather(ex_v, [jnp.full((L,), 0, jnp.int32) + r])
            exs[r, :] = spl
            for c in range(H // L):
                rows[r, pl.ds(c * L, L)] = rows[r, pl.ds(c * L, L)] * spl
            return carry2
        lax.fori_loop(0, C, _scale, 0)
        pltpu.sync_copy(rows, acc_s.at[dst_c], add=True)
        pltpu.sync_copy(exs, den_s.at[dst_c], add=True)
        return carry
    lax.fori_loop(0, NCH, _chunk, 0)

    plsc.subcore_barrier()

    def _writeback(nrows):
        pltpu.sync_copy(acc_s.at[pl.ds(row0, nrows)],
                        acc_hbm.at[cid, pl.ds(row0, nrows)])
        pltpu.sync_copy(den_s.at[pl.ds(row0, nrows)],
                        den_hbm.at[cid, pl.ds(row0, nrows)])

    pl.when(sid < NS - 1)(lambda: _writeback(RPT))
    pl.when(sid == NS - 1)(lambda: _writeback(RPT_LAST))


def _edge_phase(h, als, ald, src, dst):
    f32 = jnp.float32
    call = pl.kernel(
        _edge_body,
        out_type=(
            jax.ShapeDtypeStruct((NC, N, H), f32),
            jax.ShapeDtypeStruct((NC, N, L), f32),
        ),
        mesh=plsc.VectorSubcoreMesh(core_axis_name="c", subcore_axis_name="s"),
        compiler_params=pltpu.CompilerParams(needs_layout_passes=False,
                                             use_tc_tiling_on_sc=False),
        scratch_types=[
            pltpu.VMEM((N,), f32),
            pltpu.VMEM((N,), f32),
            pltpu.VMEM((C,), jnp.int32),
            pltpu.VMEM((C,), jnp.int32),
            pltpu.VMEM((C,), f32),
            pltpu.VMEM((C, L), f32),
            pltpu.VMEM((C, H), f32),
            pltpu.VMEM_SHARED((N, H), f32),
            pltpu.VMEM_SHARED((N, L), f32),
        ],
    )
    return call(h, als, ald, src, dst)


def _prologue_body(xd_ref, xt_ref, ws_ref, wd_ref, asrc_ref, adst_ref,
                   h_ref, als_ref, ald_ref):
    h_src = jnp.dot(xd_ref[...], ws_ref[...], preferred_element_type=jnp.float32)
    h_ref[...] = h_src
    als_ref[...] = jnp.dot(h_src, asrc_ref[...], preferred_element_type=jnp.float32)
    h_dst = jnp.dot(xt_ref[...], wd_ref[...], preferred_element_type=jnp.float32)
    ald_ref[...] = jnp.dot(h_dst, adst_ref[...], preferred_element_type=jnp.float32)


def _prologue(x_data, x_tasks, Wd_src, Wd_dst, ad_src, ad_dst):
    return pl.pallas_call(
        _prologue_body,
        out_shape=(
            jax.ShapeDtypeStruct((N, H), jnp.float32),
            jax.ShapeDtypeStruct((N,), jnp.float32),
            jax.ShapeDtypeStruct((N,), jnp.float32),
        ),
    )(x_data, x_tasks, Wd_src, Wd_dst, ad_src, ad_dst)


def _finalize(acc_ref, den_ref):
    t = (acc_ref[0] + acc_ref[1]) / (den_ref[0, :, 0] + den_ref[1, :, 0] + 1e-16)[:, None]
    return jnp.where(t > 0, t, jnp.exp(t) - 1.0)


def _mid_body(acc_ref, den_ref, w_ref, asrc_ref, adst_ref,
              h_ref, als_ref, ald_ref):
    t = _finalize(acc_ref, den_ref)
    h = jnp.dot(t, w_ref[...], preferred_element_type=jnp.float32)
    h_ref[...] = h
    als_ref[...] = jnp.dot(h, asrc_ref[...], preferred_element_type=jnp.float32)
    ald_ref[...] = jnp.dot(h, adst_ref[...], preferred_element_type=jnp.float32)


def _mid(acc, den, W, a_src, a_dst):
    return pl.pallas_call(
        _mid_body,
        out_shape=(
            jax.ShapeDtypeStruct((N, H), jnp.float32),
            jax.ShapeDtypeStruct((N,), jnp.float32),
            jax.ShapeDtypeStruct((N,), jnp.float32),
        ),
    )(acc, den, W, a_src, a_dst)


def _epilogue_body(acc_ref, den_ref, xdev_ref, wdev_ref, counts_ref, out_ref):
    t = _finalize(acc_ref, den_ref)
    out_ref[0:H] = t[0]
    cnt = jnp.maximum(counts_ref[0, 0], 1.0)
    out_ref[H:2 * H] = jnp.sum(t, axis=0) / cnt
    dev = jnp.maximum(
        jnp.dot(xdev_ref[...], wdev_ref[...], preferred_element_type=jnp.float32), 0.0)
    out_ref[2 * H:] = dev.reshape(-1)


def _epilogue(acc, den, x_devices, W_dev, counts):
    ndev = x_devices.shape[0]
    return pl.pallas_call(
        _epilogue_body,
        out_shape=jax.ShapeDtypeStruct((2 * H + ndev * H,), jnp.float32),
    )(acc, den, x_devices, W_dev, counts)


def kernel(x_data, x_tasks, x_devices, counts, edge_index_dt, edge_index_tt,
           Wd_src, Wd_dst, ad_src, ad_dst,
           W1, a1_src, a1_dst,
           W2, a2_src, a2_dst,
           W_dev):
    src_dt = edge_index_dt[0].astype(jnp.int32)
    dst_dt = edge_index_dt[1].astype(jnp.int32)
    src_tt = edge_index_tt[0].astype(jnp.int32)
    dst_tt = edge_index_tt[1].astype(jnp.int32)

    h, als, ald = _prologue(x_data, x_tasks, Wd_src, Wd_dst, ad_src, ad_dst)
    acc, den = _edge_phase(h, als, ald, src_dt, dst_dt)
    h, als, ald = _mid(acc, den, W1, a1_src, a1_dst)
    acc, den = _edge_phase(h, als, ald, src_tt, dst_tt)
    h, als, ald = _mid(acc, den, W2, a2_src, a2_dst)
    acc, den = _edge_phase(h, als, ald, src_tt, dst_tt)
    return _epilogue(acc, den, x_devices, W_dev, counts)


# speedup vs baseline: 37.5209x; 1.8246x over previous
import functools
import jax
import jax.numpy as jnp
from jax import lax
from jax.experimental import pallas as pl
from jax.experimental.pallas import tpu as pltpu
from jax.experimental.pallas import tpu_sc as plsc

H = 128
N = 10000
E = 320000

NC = 2
NS = 16
NW = NC * NS
L = 16

C = 80
EPW = E // NW
NCH = EPW // C
RPT = 624
RPT_LAST = N - (NS - 1) * RPT


def _edge_body(h_hbm, tab_hbm, src_hbm, dst_hbm,
               acc_hbm, den_hbm,
               tab_l,
               src0, src1, src2, dst0, dst1, dst2, exv0, exv1, exv2,
               rows0, rows1, rows2,
               acc_s, den_s,
               gs0, gs1, gs2, sr0, sr1, sr2, sd0, sd1, sd2):
    cid = lax.axis_index("c")
    sid = lax.axis_index("s")
    wid = sid * NC + cid
    zeros16 = jnp.zeros((L,), jnp.float32)

    BUF = ((src0, dst0, exv0, rows0, gs0, sr0, sd0),
           (src1, dst1, exv1, rows1, gs1, sr1, sd1),
           (src2, dst2, exv2, rows2, gs2, sr2, sd2))

    def _zero_row(i, carry):
        for c in range(H // L):
            rows0[i, pl.ds(c * L, L)] = zeros16
        return carry
    lax.fori_loop(0, C, _zero_row, 0)
    for k in range(C // L):
        exv0[pl.ds(k * L, L)] = zeros16

    row0 = sid * RPT

    def _zero_rows(nrows):
        full = nrows // C
        for k in range(full):
            pltpu.sync_copy(rows0.at[pl.ds(0, C)],
                            acc_s.at[pl.ds(row0 + C * k, C)])
            pltpu.sync_copy(exv0, den_s.at[pl.ds(row0 + C * k, C)])
        rem = nrows - full * C
        if rem:
            pltpu.sync_copy(rows0.at[pl.ds(0, rem)],
                            acc_s.at[pl.ds(row0 + full * C, rem)])
            pltpu.sync_copy(exv0.at[pl.ds(0, rem)],
                            den_s.at[pl.ds(row0 + full * C, rem)])

    pl.when(sid < NS - 1)(lambda: _zero_rows(RPT))
    pl.when(sid == NS - 1)(lambda: _zero_rows(RPT_LAST))

    pltpu.sync_copy(tab_hbm, tab_l)

    plsc.subcore_barrier()

    def _load_src(b, g):
        pltpu.sync_copy(src_hbm.at[pl.ds(wid * EPW + g * C, C)], BUF[b][0])

    def _load_dst(b, g):
        pltpu.sync_copy(dst_hbm.at[pl.ds(wid * EPW + g * C, C)], BUF[b][1])

    def _fire_gather(b):
        pltpu.async_copy(h_hbm.at[BUF[b][0]], BUF[b][3], BUF[b][4])

    def _wait_gather(b):
        pltpu.make_async_copy(h_hbm.at[BUF[b][0]], BUF[b][3], BUF[b][4]).wait()

    def _compute_ex(b):
        buf = BUF[b]
        for gg in range(C // L):
            sidx = buf[0][pl.ds(gg * L, L)]
            didx = buf[1][pl.ds(gg * L, L)]
            ga = plsc.unpack(plsc.bitcast(plsc.load_gather(tab_l, [sidx]),
                                          jnp.bfloat16),
                             format=plsc.PackFormat.INTERLEAVED)
            gd = plsc.unpack(plsc.bitcast(plsc.load_gather(tab_l, [didx]),
                                          jnp.bfloat16),
                             format=plsc.PackFormat.INTERLEAVED)
            s = ga[0].astype(jnp.float32) + gd[1].astype(jnp.float32)
            e = jnp.maximum(s, 0.2 * s)
            buf[2][pl.ds(gg * L, L)] = jnp.exp(e)

    def _scale(b):
        rows, exv = BUF[b][3], BUF[b][2]

        def _srow(r, carry2):
            spl = plsc.load_gather(exv, [jnp.full((L,), 0, jnp.int32) + r])
            for c in range(H // L):
                rows[r, pl.ds(c * L, L)] = rows[r, pl.ds(c * L, L)] * spl
            return carry2
        lax.fori_loop(0, C, _srow, 0)

    def _fire_scatters(b):
        buf = BUF[b]
        pltpu.async_copy(buf[3], acc_s.at[buf[1]], buf[5], add=True)
        pltpu.async_copy(buf[2], den_s.at[buf[1]], buf[6], add=True)

    def _wait_scatters(b):
        buf = BUF[b]
        pltpu.make_async_copy(buf[3], acc_s.at[buf[1]], buf[5]).wait()
        pltpu.make_async_copy(buf[2], den_s.at[buf[1]], buf[6]).wait()

    for g0 in range(2):
        _load_src(g0, g0)
        _load_dst(g0, g0)
        _fire_gather(g0)

    def _iter(a, z, g):
        _compute_ex(a)
        _wait_gather(a)
        _scale(a)
        _fire_scatters(a)
        pl.when(g + 2 < NCH)(lambda: _load_src(z, g + 2))
        pl.when(g >= 1)(lambda: _wait_scatters(z))

        def _next_gather():
            _load_dst(z, g + 2)
            _fire_gather(z)
        pl.when(g + 2 < NCH)(_next_gather)

    def _chunk(g, carry):
        for b in range(3):
            pl.when(g % 3 == b)(lambda b=b: _iter(b, (b + 2) % 3, g))
        return carry
    lax.fori_loop(0, NCH, _chunk, 0)

    _wait_scatters((NCH - 1) % 3)

    plsc.subcore_barrier()

    def _writeback(nrows):
        pltpu.sync_copy(acc_s.at[pl.ds(row0, nrows)],
                        acc_hbm.at[cid, pl.ds(row0, nrows)])
        pltpu.sync_copy(den_s.at[pl.ds(row0, nrows)],
                        den_hbm.at[cid, pl.ds(row0, nrows)])

    pl.when(sid < NS - 1)(lambda: _writeback(RPT))
    pl.when(sid == NS - 1)(lambda: _writeback(RPT_LAST))


def _edge_phase(h, tab, src, dst):
    f32 = jnp.float32
    i32 = jnp.int32
    call = pl.kernel(
        _edge_body,
        out_type=(
            jax.ShapeDtypeStruct((NC, N, H), f32),
            jax.ShapeDtypeStruct((NC, N), f32),
        ),
        mesh=plsc.VectorSubcoreMesh(core_axis_name="c", subcore_axis_name="s"),
        compiler_params=pltpu.CompilerParams(needs_layout_passes=False,
                                             use_tc_tiling_on_sc=False),
        scratch_types=[
            pltpu.VMEM((N,), i32),
            pltpu.VMEM((C,), i32),
            pltpu.VMEM((C,), i32),
            pltpu.VMEM((C,), i32),
            pltpu.VMEM((C,), i32),
            pltpu.VMEM((C,), i32),
            pltpu.VMEM((C,), i32),
            pltpu.VMEM((C,), f32),
            pltpu.VMEM((C,), f32),
            pltpu.VMEM((C,), f32),
            pltpu.VMEM((C, H), f32),
            pltpu.VMEM((C, H), f32),
            pltpu.VMEM((C, H), f32),
            pltpu.VMEM_SHARED((N, H), f32),
            pltpu.VMEM_SHARED((N,), f32),
            pltpu.SemaphoreType.DMA,
            pltpu.SemaphoreType.DMA,
            pltpu.SemaphoreType.DMA,
            pltpu.SemaphoreType.DMA,
            pltpu.SemaphoreType.DMA,
            pltpu.SemaphoreType.DMA,
            pltpu.SemaphoreType.DMA,
            pltpu.SemaphoreType.DMA,
            pltpu.SemaphoreType.DMA,
        ],
    )
    return call(h, tab, src, dst)


def _pack_logits(als, ald):
    pair = jnp.stack([als.astype(jnp.bfloat16), ald.astype(jnp.bfloat16)],
                     axis=-1)
    return jax.lax.bitcast_convert_type(pair, jnp.int32)


def _prologue_body(xd_ref, xt_ref, ws_ref, wd_ref, asrc_ref, adst_ref,
                   h_ref, als_ref, ald_ref):
    h_src = jnp.dot(xd_ref[...], ws_ref[...], preferred_element_type=jnp.float32)
    h_ref[...] = h_src
    als_ref[...] = jnp.dot(h_src, asrc_ref[...], preferred_element_type=jnp.float32)
    h_dst = jnp.dot(xt_ref[...], wd_ref[...], preferred_element_type=jnp.float32)
    ald_ref[...] = jnp.dot(h_dst, adst_ref[...], preferred_element_type=jnp.float32)


def _prologue(x_data, x_tasks, Wd_src, Wd_dst, ad_src, ad_dst):
    return pl.pallas_call(
        _prologue_body,
        out_shape=(
            jax.ShapeDtypeStruct((N, H), jnp.float32),
            jax.ShapeDtypeStruct((N,), jnp.float32),
            jax.ShapeDtypeStruct((N,), jnp.float32),
        ),
    )(x_data, x_tasks, Wd_src, Wd_dst, ad_src, ad_dst)


def _finalize(acc_ref, den_ref):
    t = (acc_ref[0] + acc_ref[1]) / (den_ref[0] + den_ref[1] + 1e-16)[:, None]
    return jnp.where(t > 0, t, jnp.exp(t) - 1.0)


def _mid_body(acc_ref, den_ref, w_ref, asrc_ref, adst_ref,
              h_ref, als_ref, ald_ref):
    t = _finalize(acc_ref, den_ref)
    h = jnp.dot(t, w_ref[...], preferred_element_type=jnp.float32)
    h_ref[...] = h
    als_ref[...] = jnp.dot(h, asrc_ref[...], preferred_element_type=jnp.float32)
    ald_ref[...] = jnp.dot(h, adst_ref[...], preferred_element_type=jnp.float32)


def _mid(acc, den, W, a_src, a_dst):
    return pl.pallas_call(
        _mid_body,
        out_shape=(
            jax.ShapeDtypeStruct((N, H), jnp.float32),
            jax.ShapeDtypeStruct((N,), jnp.float32),
            jax.ShapeDtypeStruct((N,), jnp.float32),
        ),
    )(acc, den, W, a_src, a_dst)


def _epilogue_body(acc_ref, den_ref, xdev_ref, wdev_ref, counts_ref, out_ref):
    t = _finalize(acc_ref, den_ref)
    out_ref[0:H] = t[0]
    cnt = jnp.maximum(counts_ref[0, 0], 1.0)
    out_ref[H:2 * H] = jnp.sum(t, axis=0) / cnt
    dev = jnp.maximum(
        jnp.dot(xdev_ref[...], wdev_ref[...], preferred_element_type=jnp.float32), 0.0)
    out_ref[2 * H:] = dev.reshape(-1)


def _epilogue(acc, den, x_devices, W_dev, counts):
    ndev = x_devices.shape[0]
    return pl.pallas_call(
        _epilogue_body,
        out_shape=jax.ShapeDtypeStruct((2 * H + ndev * H,), jnp.float32),
    )(acc, den, x_devices, W_dev, counts)


def kernel(x_data, x_tasks, x_devices, counts, edge_index_dt, edge_index_tt,
           Wd_src, Wd_dst, ad_src, ad_dst,
           W1, a1_src, a1_dst,
           W2, a2_src, a2_dst,
           W_dev):
    src_dt = edge_index_dt[0].astype(jnp.int32)
    dst_dt = edge_index_dt[1].astype(jnp.int32)
    src_tt = edge_index_tt[0].astype(jnp.int32)
    dst_tt = edge_index_tt[1].astype(jnp.int32)

    h, als, ald = _prologue(x_data, x_tasks, Wd_src, Wd_dst, ad_src, ad_dst)
    acc, den = _edge_phase(h, _pack_logits(als, ald), src_dt, dst_dt)
    h, als, ald = _mid(acc, den, W1, a1_src, a1_dst)
    acc, den = _edge_phase(h, _pack_logits(als, ald), src_tt, dst_tt)
    h, als, ald = _mid(acc, den, W2, a2_src, a2_dst)
    acc, den = _edge_phase(h, _pack_logits(als, ald), src_tt, dst_tt)
    return _epilogue(acc, den, x_devices, W_dev, counts)


# speedup vs baseline: 45.3355x; 1.2083x over previous
import functools
import jax
import jax.numpy as jnp
from jax import lax
from jax.experimental import pallas as pl
from jax.experimental.pallas import tpu as pltpu
from jax.experimental.pallas import tpu_sc as plsc

H = 128
N = 10000
E = 320000

NC = 2
NS = 16
NW = NC * NS
L = 16

C = 80
EPW = E // NW
NCH = EPW // C
RPT = 624
RPT_LAST = N - (NS - 1) * RPT


def _edge_body(h_hbm, tab_hbm, src_hbm, dst_hbm,
               acc_hbm, den_hbm,
               tab_l,
               src0, src1, src2, dst0, dst1, dst2, exv0, exv1, exv2,
               rows0, rows1, rows2,
               acc_s, den_s,
               gs0, gs1, gs2, sr0, sr1, sr2, sd0, sd1, sd2):
    cid = lax.axis_index("c")
    sid = lax.axis_index("s")
    wid = sid * NC + cid
    zeros16 = jnp.zeros((L,), jnp.float32)

    BUF = ((src0, dst0, exv0, rows0, gs0, sr0, sd0),
           (src1, dst1, exv1, rows1, gs1, sr1, sd1),
           (src2, dst2, exv2, rows2, gs2, sr2, sd2))

    def _zero_row(i, carry):
        for c in range(H // L):
            rows0[i, pl.ds(c * L, L)] = zeros16
        return carry
    lax.fori_loop(0, C, _zero_row, 0)
    for k in range(C // L):
        exv0[pl.ds(k * L, L)] = zeros16

    row0 = sid * RPT

    def _zero_rows(nrows):
        full = nrows // C
        for k in range(full):
            pltpu.sync_copy(rows0.at[pl.ds(0, C)],
                            acc_s.at[pl.ds(row0 + C * k, C)])
            pltpu.sync_copy(exv0, den_s.at[pl.ds(row0 + C * k, C)])
        rem = nrows - full * C
        if rem:
            pltpu.sync_copy(rows0.at[pl.ds(0, rem)],
                            acc_s.at[pl.ds(row0 + full * C, rem)])
            pltpu.sync_copy(exv0.at[pl.ds(0, rem)],
                            den_s.at[pl.ds(row0 + full * C, rem)])

    pl.when(sid < NS - 1)(lambda: _zero_rows(RPT))
    pl.when(sid == NS - 1)(lambda: _zero_rows(RPT_LAST))

    pltpu.sync_copy(tab_hbm, tab_l)

    plsc.subcore_barrier()

    def _load_src(b, g):
        pltpu.sync_copy(src_hbm.at[pl.ds(wid * EPW + g * C, C)], BUF[b][0])

    def _load_dst(b, g):
        pltpu.sync_copy(dst_hbm.at[pl.ds(wid * EPW + g * C, C)], BUF[b][1])

    def _fire_gather(b):
        pltpu.async_copy(h_hbm.at[BUF[b][0]], BUF[b][3], BUF[b][4])

    def _wait_gather(b):
        pltpu.make_async_copy(h_hbm.at[BUF[b][0]], BUF[b][3], BUF[b][4]).wait()

    def _compute_ex(b):
        buf = BUF[b]
        for gg in range(C // L):
            sidx = buf[0][pl.ds(gg * L, L)]
            didx = buf[1][pl.ds(gg * L, L)]
            ga = plsc.unpack(plsc.bitcast(plsc.load_gather(tab_l, [sidx]),
                                          jnp.bfloat16),
                             format=plsc.PackFormat.INTERLEAVED)
            gd = plsc.unpack(plsc.bitcast(plsc.load_gather(tab_l, [didx]),
                                          jnp.bfloat16),
                             format=plsc.PackFormat.INTERLEAVED)
            s = ga[0].astype(jnp.float32) + gd[1].astype(jnp.float32)
            e = jnp.maximum(s, 0.2 * s)
            buf[2][pl.ds(gg * L, L)] = jnp.exp(e)

    def _scale(b):
        rows, exv = BUF[b][3], BUF[b][2]

        @plsc.parallel_loop(0, C, 1, unroll=8)
        def _srow(r):
            spl = plsc.load_gather(exv, [jnp.full((L,), 0, jnp.int32) + r])
            for c in range(H // L):
                rows[r, pl.ds(c * L, L)] = rows[r, pl.ds(c * L, L)] * spl

    def _fire_scatters(b):
        buf = BUF[b]
        pltpu.async_copy(buf[3], acc_s.at[buf[1]], buf[5], add=True)
        pltpu.async_copy(buf[2], den_s.at[buf[1]], buf[6], add=True)

    def _wait_scatters(b):
        buf = BUF[b]
        pltpu.make_async_copy(buf[3], acc_s.at[buf[1]], buf[5]).wait()
        pltpu.make_async_copy(buf[2], den_s.at[buf[1]], buf[6]).wait()

    for g0 in range(2):
        _load_src(g0, g0)
        _load_dst(g0, g0)
        _fire_gather(g0)

    def _iter(a, z, g):
        _compute_ex(a)
        _wait_gather(a)
        _scale(a)
        _fire_scatters(a)
        pl.when(g + 2 < NCH)(lambda: _load_src(z, g + 2))
        pl.when(g >= 1)(lambda: _wait_scatters(z))

        def _next_gather():
            _load_dst(z, g + 2)
            _fire_gather(z)
        pl.when(g + 2 < NCH)(_next_gather)

    def _chunk(g, carry):
        for b in range(3):
            pl.when(g % 3 == b)(lambda b=b: _iter(b, (b + 2) % 3, g))
        return carry
    lax.fori_loop(0, NCH, _chunk, 0)

    _wait_scatters((NCH - 1) % 3)

    plsc.subcore_barrier()

    def _writeback(nrows):
        pltpu.sync_copy(acc_s.at[pl.ds(row0, nrows)],
                        acc_hbm.at[cid, pl.ds(row0, nrows)])
        pltpu.sync_copy(den_s.at[pl.ds(row0, nrows)],
                        den_hbm.at[cid, pl.ds(row0, nrows)])

    pl.when(sid < NS - 1)(lambda: _writeback(RPT))
    pl.when(sid == NS - 1)(lambda: _writeback(RPT_LAST))


def _edge_phase(h, tab, src, dst):
    f32 = jnp.float32
    i32 = jnp.int32
    call = pl.kernel(
        _edge_body,
        out_type=(
            jax.ShapeDtypeStruct((NC, N, H), f32),
            jax.ShapeDtypeStruct((NC, N), f32),
        ),
        mesh=plsc.VectorSubcoreMesh(core_axis_name="c", subcore_axis_name="s"),
        compiler_params=pltpu.CompilerParams(needs_layout_passes=False,
                                             use_tc_tiling_on_sc=False),
        scratch_types=[
            pltpu.VMEM((N,), i32),
            pltpu.VMEM((C,), i32),
            pltpu.VMEM((C,), i32),
            pltpu.VMEM((C,), i32),
            pltpu.VMEM((C,), i32),
            pltpu.VMEM((C,), i32),
            pltpu.VMEM((C,), i32),
            pltpu.VMEM((C,), f32),
            pltpu.VMEM((C,), f32),
            pltpu.VMEM((C,), f32),
            pltpu.VMEM((C, H), f32),
            pltpu.VMEM((C, H), f32),
            pltpu.VMEM((C, H), f32),
            pltpu.VMEM_SHARED((N, H), f32),
            pltpu.VMEM_SHARED((N,), f32),
            pltpu.SemaphoreType.DMA,
            pltpu.SemaphoreType.DMA,
            pltpu.SemaphoreType.DMA,
            pltpu.SemaphoreType.DMA,
            pltpu.SemaphoreType.DMA,
            pltpu.SemaphoreType.DMA,
            pltpu.SemaphoreType.DMA,
            pltpu.SemaphoreType.DMA,
            pltpu.SemaphoreType.DMA,
        ],
    )
    return call(h, tab, src, dst)


def _pack_logits(als, ald):
    pair = jnp.stack([als.astype(jnp.bfloat16), ald.astype(jnp.bfloat16)],
                     axis=-1)
    return jax.lax.bitcast_convert_type(pair, jnp.int32)


def _prologue_body(xd_ref, xt_ref, ws_ref, wd_ref, asrc_ref, adst_ref,
                   h_ref, als_ref, ald_ref):
    h_src = jnp.dot(xd_ref[...], ws_ref[...], preferred_element_type=jnp.float32)
    h_ref[...] = h_src
    als_ref[...] = jnp.dot(h_src, asrc_ref[...], preferred_element_type=jnp.float32)
    h_dst = jnp.dot(xt_ref[...], wd_ref[...], preferred_element_type=jnp.float32)
    ald_ref[...] = jnp.dot(h_dst, adst_ref[...], preferred_element_type=jnp.float32)


def _prologue(x_data, x_tasks, Wd_src, Wd_dst, ad_src, ad_dst):
    return pl.pallas_call(
        _prologue_body,
        out_shape=(
            jax.ShapeDtypeStruct((N, H), jnp.float32),
            jax.ShapeDtypeStruct((N,), jnp.float32),
            jax.ShapeDtypeStruct((N,), jnp.float32),
        ),
    )(x_data, x_tasks, Wd_src, Wd_dst, ad_src, ad_dst)


def _finalize(acc_ref, den_ref):
    t = (acc_ref[0] + acc_ref[1]) / (den_ref[0] + den_ref[1] + 1e-16)[:, None]
    return jnp.where(t > 0, t, jnp.exp(t) - 1.0)


def _mid_body(acc_ref, den_ref, w_ref, asrc_ref, adst_ref,
              h_ref, als_ref, ald_ref):
    t = _finalize(acc_ref, den_ref)
    h = jnp.dot(t, w_ref[...], preferred_element_type=jnp.float32)
    h_ref[...] = h
    als_ref[...] = jnp.dot(h, asrc_ref[...], preferred_element_type=jnp.float32)
    ald_ref[...] = jnp.dot(h, adst_ref[...], preferred_element_type=jnp.float32)


def _mid(acc, den, W, a_src, a_dst):
    return pl.pallas_call(
        _mid_body,
        out_shape=(
            jax.ShapeDtypeStruct((N, H), jnp.float32),
            jax.ShapeDtypeStruct((N,), jnp.float32),
            jax.ShapeDtypeStruct((N,), jnp.float32),
        ),
    )(acc, den, W, a_src, a_dst)


def _epilogue_body(acc_ref, den_ref, xdev_ref, wdev_ref, counts_ref, out_ref):
    t = _finalize(acc_ref, den_ref)
    out_ref[0:H] = t[0]
    cnt = jnp.maximum(counts_ref[0, 0], 1.0)
    out_ref[H:2 * H] = jnp.sum(t, axis=0) / cnt
    dev = jnp.maximum(
        jnp.dot(xdev_ref[...], wdev_ref[...], preferred_element_type=jnp.float32), 0.0)
    out_ref[2 * H:] = dev.reshape(-1)


def _epilogue(acc, den, x_devices, W_dev, counts):
    ndev = x_devices.shape[0]
    return pl.pallas_call(
        _epilogue_body,
        out_shape=jax.ShapeDtypeStruct((2 * H + ndev * H,), jnp.float32),
    )(acc, den, x_devices, W_dev, counts)


def kernel(x_data, x_tasks, x_devices, counts, edge_index_dt, edge_index_tt,
           Wd_src, Wd_dst, ad_src, ad_dst,
           W1, a1_src, a1_dst,
           W2, a2_src, a2_dst,
           W_dev):
    src_dt = edge_index_dt[0].astype(jnp.int32)
    dst_dt = edge_index_dt[1].astype(jnp.int32)
    src_tt = edge_index_tt[0].astype(jnp.int32)
    dst_tt = edge_index_tt[1].astype(jnp.int32)

    h, als, ald = _prologue(x_data, x_tasks, Wd_src, Wd_dst, ad_src, ad_dst)
    acc, den = _edge_phase(h, _pack_logits(als, ald), src_dt, dst_dt)
    h, als, ald = _mid(acc, den, W1, a1_src, a1_dst)
    acc, den = _edge_phase(h, _pack_logits(als, ald), src_tt, dst_tt)
    h, als, ald = _mid(acc, den, W2, a2_src, a2_dst)
    acc, den = _edge_phase(h, _pack_logits(als, ald), src_tt, dst_tt)
    return _epilogue(acc, den, x_devices, W_dev, counts)


# speedup vs baseline: 46.6126x; 1.0282x over previous
import functools
import jax
import jax.numpy as jnp
from jax import lax
from jax.experimental import pallas as pl
from jax.experimental.pallas import tpu as pltpu
from jax.experimental.pallas import tpu_sc as plsc

H = 128
N = 10000
E = 320000

NC = 2
NS = 16
NW = NC * NS
L = 16

C = 80
EPW = E // NW
NCH = EPW // C
RPT = 624
RPT_LAST = N - (NS - 1) * RPT


def _edge_body(h_hbm, tab_hbm, ei_hbm,
               acc_hbm, den_hbm,
               tab_l,
               src0, src1, src2, dst0, dst1, dst2, exv0, exv1, exv2,
               rows0, rows1, rows2,
               acc_s, den_s,
               gs0, gs1, gs2, sr0, sr1, sr2, sd0, sd1, sd2):
    cid = lax.axis_index("c")
    sid = lax.axis_index("s")
    wid = sid * NC + cid
    zeros16 = jnp.zeros((L,), jnp.float32)

    BUF = ((src0, dst0, exv0, rows0, gs0, sr0, sd0),
           (src1, dst1, exv1, rows1, gs1, sr1, sd1),
           (src2, dst2, exv2, rows2, gs2, sr2, sd2))

    def _zero_row(i, carry):
        for c in range(H // L):
            rows0[i, pl.ds(c * L, L)] = zeros16
        return carry
    lax.fori_loop(0, C, _zero_row, 0)
    for k in range(C // L):
        exv0[pl.ds(k * L, L)] = zeros16

    row0 = sid * RPT

    def _zero_rows(nrows):
        full = nrows // C
        for k in range(full):
            pltpu.sync_copy(rows0.at[pl.ds(0, C)],
                            acc_s.at[pl.ds(row0 + C * k, C)])
            pltpu.sync_copy(exv0, den_s.at[pl.ds(row0 + C * k, C)])
        rem = nrows - full * C
        if rem:
            pltpu.sync_copy(rows0.at[pl.ds(0, rem)],
                            acc_s.at[pl.ds(row0 + full * C, rem)])
            pltpu.sync_copy(exv0.at[pl.ds(0, rem)],
                            den_s.at[pl.ds(row0 + full * C, rem)])

    pl.when(sid < NS - 1)(lambda: _zero_rows(RPT))
    pl.when(sid == NS - 1)(lambda: _zero_rows(RPT_LAST))

    pltpu.sync_copy(tab_hbm, tab_l)

    plsc.subcore_barrier()

    def _load_src(b, g):
        pltpu.sync_copy(ei_hbm.at[0, pl.ds(wid * EPW + g * C, C)], BUF[b][0])

    def _load_dst(b, g):
        pltpu.sync_copy(ei_hbm.at[1, pl.ds(wid * EPW + g * C, C)], BUF[b][1])

    def _fire_gather(b):
        pltpu.async_copy(h_hbm.at[BUF[b][0]], BUF[b][3], BUF[b][4])

    def _wait_gather(b):
        pltpu.make_async_copy(h_hbm.at[BUF[b][0]], BUF[b][3], BUF[b][4]).wait()

    def _compute_ex(b):
        buf = BUF[b]
        for gg in range(C // L):
            sidx = buf[0][pl.ds(gg * L, L)]
            didx = buf[1][pl.ds(gg * L, L)]
            ga = plsc.unpack(plsc.bitcast(plsc.load_gather(tab_l, [sidx]),
                                          jnp.bfloat16),
                             format=plsc.PackFormat.INTERLEAVED)
            gd = plsc.unpack(plsc.bitcast(plsc.load_gather(tab_l, [didx]),
                                          jnp.bfloat16),
                             format=plsc.PackFormat.INTERLEAVED)
            s = ga[0].astype(jnp.float32) + gd[1].astype(jnp.float32)
            e = jnp.maximum(s, 0.2 * s)
            buf[2][pl.ds(gg * L, L)] = jnp.exp(e)

    def _scale(b):
        rows, exv = BUF[b][3], BUF[b][2]

        @plsc.parallel_loop(0, C, 1, unroll=16)
        def _srow(r):
            spl = plsc.load_gather(exv, [jnp.full((L,), 0, jnp.int32) + r])
            for c in range(H // L):
                rows[r, pl.ds(c * L, L)] = rows[r, pl.ds(c * L, L)] * spl

    def _fire_scatters(b):
        buf = BUF[b]
        pltpu.async_copy(buf[3], acc_s.at[buf[1]], buf[5], add=True)
        pltpu.async_copy(buf[2], den_s.at[buf[1]], buf[6], add=True)

    def _wait_scatters(b):
        buf = BUF[b]
        pltpu.make_async_copy(buf[3], acc_s.at[buf[1]], buf[5]).wait()
        pltpu.make_async_copy(buf[2], den_s.at[buf[1]], buf[6]).wait()

    for g0 in range(2):
        _load_src(g0, g0)
        _load_dst(g0, g0)
        _fire_gather(g0)

    def _iter(a, z, g):
        _compute_ex(a)
        _wait_gather(a)
        _scale(a)
        _fire_scatters(a)
        pl.when(g + 2 < NCH)(lambda: _load_src(z, g + 2))
        pl.when(g >= 1)(lambda: _wait_scatters(z))

        def _next_gather():
            _load_dst(z, g + 2)
            _fire_gather(z)
        pl.when(g + 2 < NCH)(_next_gather)

    def _chunk(g, carry):
        for b in range(3):
            pl.when(g % 3 == b)(lambda b=b: _iter(b, (b + 2) % 3, g))
        return carry
    lax.fori_loop(0, NCH, _chunk, 0)

    _wait_scatters((NCH - 1) % 3)

    plsc.subcore_barrier()

    def _writeback(nrows):
        pltpu.sync_copy(acc_s.at[pl.ds(row0, nrows)],
                        acc_hbm.at[cid, pl.ds(row0, nrows)])
        pltpu.sync_copy(den_s.at[pl.ds(row0, nrows)],
                        den_hbm.at[cid, pl.ds(row0, nrows)])

    pl.when(sid < NS - 1)(lambda: _writeback(RPT))
    pl.when(sid == NS - 1)(lambda: _writeback(RPT_LAST))


def _edge_phase(h, tab, edge_index):
    f32 = jnp.float32
    i32 = jnp.int32
    call = pl.kernel(
        _edge_body,
        out_type=(
            jax.ShapeDtypeStruct((NC, N, H), f32),
            jax.ShapeDtypeStruct((NC, N), f32),
        ),
        mesh=plsc.VectorSubcoreMesh(core_axis_name="c", subcore_axis_name="s"),
        compiler_params=pltpu.CompilerParams(needs_layout_passes=False,
                                             use_tc_tiling_on_sc=False),
        scratch_types=[
            pltpu.VMEM((N,), i32),
            pltpu.VMEM((C,), i32),
            pltpu.VMEM((C,), i32),
            pltpu.VMEM((C,), i32),
            pltpu.VMEM((C,), i32),
            pltpu.VMEM((C,), i32),
            pltpu.VMEM((C,), i32),
            pltpu.VMEM((C,), f32),
            pltpu.VMEM((C,), f32),
            pltpu.VMEM((C,), f32),
            pltpu.VMEM((C, H), f32),
            pltpu.VMEM((C, H), f32),
            pltpu.VMEM((C, H), f32),
            pltpu.VMEM_SHARED((N, H), f32),
            pltpu.VMEM_SHARED((N,), f32),
            pltpu.SemaphoreType.DMA,
            pltpu.SemaphoreType.DMA,
            pltpu.SemaphoreType.DMA,
            pltpu.SemaphoreType.DMA,
            pltpu.SemaphoreType.DMA,
            pltpu.SemaphoreType.DMA,
            pltpu.SemaphoreType.DMA,
            pltpu.SemaphoreType.DMA,
            pltpu.SemaphoreType.DMA,
        ],
    )
    return call(h, tab, edge_index)


def _pack_logits_tc(als, ald):
    ab = jax.lax.bitcast_convert_type(als, jnp.int32)
    db = jax.lax.bitcast_convert_type(ald, jnp.int32)
    ar = (ab + 0x7FFF + ((ab >> 16) & 1)) >> 16
    dr = (db + 0x7FFF + ((db >> 16) & 1)) >> 16
    return (dr << 16) | (ar & 0xFFFF)


def _prologue_body(xd_ref, xt_ref, ws_ref, wd_ref, asrc_ref, adst_ref,
                   h_ref, tab_ref):
    h_src = jnp.dot(xd_ref[...], ws_ref[...], preferred_element_type=jnp.float32)
    h_ref[...] = h_src
    als = jnp.dot(h_src, asrc_ref[...], preferred_element_type=jnp.float32)
    h_dst = jnp.dot(xt_ref[...], wd_ref[...], preferred_element_type=jnp.float32)
    ald = jnp.dot(h_dst, adst_ref[...], preferred_element_type=jnp.float32)
    tab_ref[...] = _pack_logits_tc(als, ald)


def _prologue(x_data, x_tasks, Wd_src, Wd_dst, ad_src, ad_dst):
    return pl.pallas_call(
        _prologue_body,
        out_shape=(
            jax.ShapeDtypeStruct((N, H), jnp.float32),
            jax.ShapeDtypeStruct((N,), jnp.int32),
        ),
    )(x_data, x_tasks, Wd_src, Wd_dst, ad_src, ad_dst)


def _finalize(acc_ref, den_ref):
    t = (acc_ref[0] + acc_ref[1]) / (den_ref[0] + den_ref[1] + 1e-16)[:, None]
    return jnp.where(t > 0, t, jnp.exp(t) - 1.0)


def _mid_body(acc_ref, den_ref, w_ref, asrc_ref, adst_ref,
              h_ref, tab_ref):
    t = _finalize(acc_ref, den_ref)
    h = jnp.dot(t, w_ref[...], preferred_element_type=jnp.float32)
    h_ref[...] = h
    als = jnp.dot(h, asrc_ref[...], preferred_element_type=jnp.float32)
    ald = jnp.dot(h, adst_ref[...], preferred_element_type=jnp.float32)
    tab_ref[...] = _pack_logits_tc(als, ald)


def _mid(acc, den, W, a_src, a_dst):
    return pl.pallas_call(
        _mid_body,
        out_shape=(
            jax.ShapeDtypeStruct((N, H), jnp.float32),
            jax.ShapeDtypeStruct((N,), jnp.int32),
        ),
    )(acc, den, W, a_src, a_dst)


def _epilogue_body(acc_ref, den_ref, xdev_ref, wdev_ref, counts_ref, out_ref):
    t = _finalize(acc_ref, den_ref)
    out_ref[0:H] = t[0]
    cnt = jnp.maximum(counts_ref[0, 0], 1.0)
    out_ref[H:2 * H] = jnp.sum(t, axis=0) / cnt
    dev = jnp.maximum(
        jnp.dot(xdev_ref[...], wdev_ref[...], preferred_element_type=jnp.float32), 0.0)
    out_ref[2 * H:] = dev.reshape(-1)


def _epilogue(acc, den, x_devices, W_dev, counts):
    ndev = x_devices.shape[0]
    return pl.pallas_call(
        _epilogue_body,
        out_shape=jax.ShapeDtypeStruct((2 * H + ndev * H,), jnp.float32),
    )(acc, den, x_devices, W_dev, counts)


def kernel(x_data, x_tasks, x_devices, counts, edge_index_dt, edge_index_tt,
           Wd_src, Wd_dst, ad_src, ad_dst,
           W1, a1_src, a1_dst,
           W2, a2_src, a2_dst,
           W_dev):
    ei_dt = edge_index_dt.astype(jnp.int32)
    ei_tt = edge_index_tt.astype(jnp.int32)

    h, tab = _prologue(x_data, x_tasks, Wd_src, Wd_dst, ad_src, ad_dst)
    acc, den = _edge_phase(h, tab, ei_dt)
    h, tab = _mid(acc, den, W1, a1_src, a1_dst)
    acc, den = _edge_phase(h, tab, ei_tt)
    h, tab = _mid(acc, den, W2, a2_src, a2_dst)
    acc, den = _edge_phase(h, tab, ei_tt)
    return _epilogue(acc, den, x_devices, W_dev, counts)


# speedup vs baseline: 68.3683x; 1.4667x over previous
import functools
import jax
import jax.numpy as jnp
from jax import lax
from jax.experimental import pallas as pl
from jax.experimental.pallas import tpu as pltpu
from jax.experimental.pallas import tpu_sc as plsc

H = 128
N = 10000
E = 320000

NC = 2
NS = 16
NW = NC * NS
L = 16

C = 80
EPW = E // NW
NCH = EPW // C
RPT = 624
RPT_LAST = N - (NS - 1) * RPT


def _edge_body(h_hbm, tab_hbm, ei_hbm,
               acc_hbm, den_hbm,
               tab_l,
               src0, src1, src2, dst0, dst1, dst2, exv0, exv1, exv2,
               rows0, rows1, rows2,
               acc_s, den_s,
               gs0, gs1, gs2, sr0, sr1, sr2, sd0, sd1, sd2,
               is0, is1, is2, id0, id1, id2):
    cid = lax.axis_index("c")
    sid = lax.axis_index("s")
    wid = sid * NC + cid
    zeros16 = jnp.zeros((L,), jnp.float32)

    BUF = ((src0, dst0, exv0, rows0, gs0, sr0, sd0, is0, id0),
           (src1, dst1, exv1, rows1, gs1, sr1, sd1, is1, id1),
           (src2, dst2, exv2, rows2, gs2, sr2, sd2, is2, id2))

    def _zero_row(i, carry):
        for c in range(H // L):
            rows0[i, pl.ds(c * L, L)] = zeros16
        return carry
    lax.fori_loop(0, C, _zero_row, 0)
    for k in range(C // L):
        exv0[pl.ds(k * L, L)] = zeros16

    row0 = sid * RPT

    def _zero_rows(nrows):
        full = nrows // C
        for k in range(full):
            pltpu.sync_copy(rows0.at[pl.ds(0, C)],
                            acc_s.at[pl.ds(row0 + C * k, C)])
            pltpu.sync_copy(exv0, den_s.at[pl.ds(row0 + C * k, C)])
        rem = nrows - full * C
        if rem:
            pltpu.sync_copy(rows0.at[pl.ds(0, rem)],
                            acc_s.at[pl.ds(row0 + full * C, rem)])
            pltpu.sync_copy(exv0.at[pl.ds(0, rem)],
                            den_s.at[pl.ds(row0 + full * C, rem)])

    pl.when(sid < NS - 1)(lambda: _zero_rows(RPT))
    pl.when(sid == NS - 1)(lambda: _zero_rows(RPT_LAST))

    pltpu.sync_copy(tab_hbm, tab_l)

    plsc.subcore_barrier()

    def _load_src(b, g):
        pltpu.sync_copy(ei_hbm.at[0, pl.ds(wid * EPW + g * C, C)], BUF[b][0])

    def _load_dst(b, g):
        pltpu.sync_copy(ei_hbm.at[1, pl.ds(wid * EPW + g * C, C)], BUF[b][1])

    def _fire_load_src(b, g):
        pltpu.async_copy(ei_hbm.at[0, pl.ds(wid * EPW + g * C, C)], BUF[b][0],
                         BUF[b][7])

    def _wait_load_src(b):
        pltpu.make_async_copy(ei_hbm.at[0, pl.ds(0, C)], BUF[b][0],
                              BUF[b][7]).wait()

    def _fire_load_dst(b, g):
        pltpu.async_copy(ei_hbm.at[1, pl.ds(wid * EPW + g * C, C)], BUF[b][1],
                         BUF[b][8])

    def _wait_load_dst(b):
        pltpu.make_async_copy(ei_hbm.at[1, pl.ds(0, C)], BUF[b][1],
                              BUF[b][8]).wait()

    def _fire_gather(b):
        pltpu.async_copy(h_hbm.at[BUF[b][0]], BUF[b][3], BUF[b][4])

    def _wait_gather(b):
        pltpu.make_async_copy(h_hbm.at[BUF[b][0]], BUF[b][3], BUF[b][4]).wait()

    def _compute_ex(b):
        buf = BUF[b]
        for gg in range(C // L):
            sidx = buf[0][pl.ds(gg * L, L)]
            didx = buf[1][pl.ds(gg * L, L)]
            ga = plsc.unpack(plsc.bitcast(plsc.load_gather(tab_l, [sidx]),
                                          jnp.bfloat16),
                             format=plsc.PackFormat.INTERLEAVED)
            gd = plsc.unpack(plsc.bitcast(plsc.load_gather(tab_l, [didx]),
                                          jnp.bfloat16),
                             format=plsc.PackFormat.INTERLEAVED)
            s = ga[0].astype(jnp.float32) + gd[1].astype(jnp.float32)
            e = jnp.maximum(s, 0.2 * s)
            buf[2][pl.ds(gg * L, L)] = jnp.exp(e)

    def _scale(b):
        rows, exv = BUF[b][3], BUF[b][2]

        @plsc.parallel_loop(0, C, 1, unroll=16)
        def _srow(r):
            spl = plsc.load_gather(exv, [jnp.full((L,), 0, jnp.int32) + r])
            for c in range(H // L):
                rows[r, pl.ds(c * L, L)] = rows[r, pl.ds(c * L, L)] * spl

    def _fire_scatters(b):
        buf = BUF[b]
        pltpu.async_copy(buf[3], acc_s.at[buf[1]], buf[5], add=True)
        pltpu.async_copy(buf[2], den_s.at[buf[1]], buf[6], add=True)

    def _wait_scatters(b):
        buf = BUF[b]
        pltpu.make_async_copy(buf[3], acc_s.at[buf[1]], buf[5]).wait()
        pltpu.make_async_copy(buf[2], den_s.at[buf[1]], buf[6]).wait()

    for g0 in range(2):
        _load_src(g0, g0)
        _load_dst(g0, g0)
        _fire_gather(g0)

    def _iter(a, z, g):
        pl.when(g + 2 < NCH)(lambda: _fire_load_src(z, g + 2))
        pl.when(g >= 2)(lambda: _wait_load_dst(a))
        _compute_ex(a)
        _wait_gather(a)
        _scale(a)
        _fire_scatters(a)
        pl.when(g >= 1)(lambda: _wait_scatters(z))

        def _next_gather():
            _fire_load_dst(z, g + 2)
            _wait_load_src(z)
            _fire_gather(z)
        pl.when(g + 2 < NCH)(_next_gather)

    def _chunk(g, carry):
        for b in range(3):
            pl.when(g % 3 == b)(lambda b=b: _iter(b, (b + 2) % 3, g))
        return carry
    lax.fori_loop(0, NCH, _chunk, 0)

    _wait_scatters((NCH - 1) % 3)

    plsc.subcore_barrier()

    def _writeback(nrows):
        pltpu.sync_copy(acc_s.at[pl.ds(row0, nrows)],
                        acc_hbm.at[cid, pl.ds(row0, nrows)])
        pltpu.sync_copy(den_s.at[pl.ds(row0, nrows)],
                        den_hbm.at[cid, pl.ds(row0, nrows)])

    pl.when(sid < NS - 1)(lambda: _writeback(RPT))
    pl.when(sid == NS - 1)(lambda: _writeback(RPT_LAST))


def _edge_phase(h, tab, edge_index):
    f32 = jnp.float32
    i32 = jnp.int32
    call = pl.kernel(
        _edge_body,
        out_type=(
            jax.ShapeDtypeStruct((NC, N, H), f32),
            jax.ShapeDtypeStruct((NC, N), f32),
        ),
        mesh=plsc.VectorSubcoreMesh(core_axis_name="c", subcore_axis_name="s"),
        compiler_params=pltpu.CompilerParams(needs_layout_passes=False,
                                             use_tc_tiling_on_sc=False),
        scratch_types=[
            pltpu.VMEM((N,), i32),
            pltpu.VMEM((C,), i32),
            pltpu.VMEM((C,), i32),
            pltpu.VMEM((C,), i32),
            pltpu.VMEM((C,), i32),
            pltpu.VMEM((C,), i32),
            pltpu.VMEM((C,), i32),
            pltpu.VMEM((C,), f32),
            pltpu.VMEM((C,), f32),
            pltpu.VMEM((C,), f32),
            pltpu.VMEM((C, H), f32),
            pltpu.VMEM((C, H), f32),
            pltpu.VMEM((C, H), f32),
            pltpu.VMEM_SHARED((N, H), f32),
            pltpu.VMEM_SHARED((N,), f32),
            pltpu.SemaphoreType.DMA,
            pltpu.SemaphoreType.DMA,
            pltpu.SemaphoreType.DMA,
            pltpu.SemaphoreType.DMA,
            pltpu.SemaphoreType.DMA,
            pltpu.SemaphoreType.DMA,
            pltpu.SemaphoreType.DMA,
            pltpu.SemaphoreType.DMA,
            pltpu.SemaphoreType.DMA,
            pltpu.SemaphoreType.DMA,
            pltpu.SemaphoreType.DMA,
            pltpu.SemaphoreType.DMA,
            pltpu.SemaphoreType.DMA,
            pltpu.SemaphoreType.DMA,
            pltpu.SemaphoreType.DMA,
        ],
    )
    return call(h, tab, edge_index)


def _pack_logits_tc(als, ald):
    ab = jax.lax.bitcast_convert_type(als, jnp.int32)
    db = jax.lax.bitcast_convert_type(ald, jnp.int32)
    ar = (ab + 0x7FFF + ((ab >> 16) & 1)) >> 16
    dr = (db + 0x7FFF + ((db >> 16) & 1)) >> 16
    return (dr << 16) | (ar & 0xFFFF)


def _prologue_body(xd_ref, xt_ref, ws_ref, wd_ref, asrc_ref, adst_ref,
                   h_ref, tab_ref):
    h_src = jnp.dot(xd_ref[...], ws_ref[...], preferred_element_type=jnp.float32)
    h_ref[...] = h_src
    als = jnp.dot(h_src, asrc_ref[...], preferred_element_type=jnp.float32)
    h_dst = jnp.dot(xt_ref[...], wd_ref[...], preferred_element_type=jnp.float32)
    ald = jnp.dot(h_dst, adst_ref[...], preferred_element_type=jnp.float32)
    tab_ref[...] = _pack_logits_tc(als, ald)


def _prologue(x_data, x_tasks, Wd_src, Wd_dst, ad_src, ad_dst):
    return pl.pallas_call(
        _prologue_body,
        out_shape=(
            jax.ShapeDtypeStruct((N, H), jnp.float32),
            jax.ShapeDtypeStruct((N,), jnp.int32),
        ),
    )(x_data, x_tasks, Wd_src, Wd_dst, ad_src, ad_dst)


def _finalize(acc_ref, den_ref):
    t = (acc_ref[0] + acc_ref[1]) / (den_ref[0] + den_ref[1] + 1e-16)[:, None]
    return jnp.where(t > 0, t, jnp.exp(t) - 1.0)


def _mid_body(acc_ref, den_ref, w_ref, asrc_ref, adst_ref,
              h_ref, tab_ref):
    t = _finalize(acc_ref, den_ref)
    h = jnp.dot(t, w_ref[...], preferred_element_type=jnp.float32)
    h_ref[...] = h
    als = jnp.dot(h, asrc_ref[...], preferred_element_type=jnp.float32)
    ald = jnp.dot(h, adst_ref[...], preferred_element_type=jnp.float32)
    tab_ref[...] = _pack_logits_tc(als, ald)


def _mid(acc, den, W, a_src, a_dst):
    return pl.pallas_call(
        _mid_body,
        out_shape=(
            jax.ShapeDtypeStruct((N, H), jnp.float32),
            jax.ShapeDtypeStruct((N,), jnp.int32),
        ),
    )(acc, den, W, a_src, a_dst)


def _epilogue_body(acc_ref, den_ref, xdev_ref, wdev_ref, counts_ref, out_ref):
    t = _finalize(acc_ref, den_ref)
    out_ref[0:H] = t[0]
    cnt = jnp.maximum(counts_ref[0, 0], 1.0)
    out_ref[H:2 * H] = jnp.sum(t, axis=0) / cnt
    dev = jnp.maximum(
        jnp.dot(xdev_ref[...], wdev_ref[...], preferred_element_type=jnp.float32), 0.0)
    out_ref[2 * H:] = dev.reshape(-1)


def _epilogue(acc, den, x_devices, W_dev, counts):
    ndev = x_devices.shape[0]
    return pl.pallas_call(
        _epilogue_body,
        out_shape=jax.ShapeDtypeStruct((2 * H + ndev * H,), jnp.float32),
    )(acc, den, x_devices, W_dev, counts)


def kernel(x_data, x_tasks, x_devices, counts, edge_index_dt, edge_index_tt,
           Wd_src, Wd_dst, ad_src, ad_dst,
           W1, a1_src, a1_dst,
           W2, a2_src, a2_dst,
           W_dev):
    ei_dt = edge_index_dt.astype(jnp.int32)
    ei_tt = edge_index_tt.astype(jnp.int32)

    h, tab = _prologue(x_data, x_tasks, Wd_src, Wd_dst, ad_src, ad_dst)
    acc, den = _edge_phase(h, tab, ei_dt)
    h, tab = _mid(acc, den, W1, a1_src, a1_dst)
    acc, den = _edge_phase(h, tab, ei_tt)
    h, tab = _mid(acc, den, W2, a2_src, a2_dst)
    acc, den = _edge_phase(h, tab, ei_tt)
    return _epilogue(acc, den, x_devices, W_dev, counts)


# speedup vs baseline: 68.4225x; 1.0008x over previous
import functools
import jax
import jax.numpy as jnp
from jax import lax
from jax.experimental import pallas as pl
from jax.experimental.pallas import tpu as pltpu
from jax.experimental.pallas import tpu_sc as plsc

H = 128
N = 10000
E = 320000

NC = 2
NS = 16
NW = NC * NS
L = 16

C = 80
EPW = E // NW
NCH = EPW // C
RPT = 624
RPT_LAST = N - (NS - 1) * RPT


def _edge_body(h_hbm, tab_hbm, ei_hbm,
               acc_hbm, den_hbm,
               tab_l,
               src0, src1, src2, dst0, dst1, dst2, exv0, exv1, exv2,
               rows0, rows1, rows2, frows0, frows1,
               acc_s, den_s,
               gs0, gs1, gs2, sr0, sr1, sr2, sd0, sd1, sd2,
               is0, is1, is2, id0, id1, id2):
    cid = lax.axis_index("c")
    sid = lax.axis_index("s")
    wid = sid * NC + cid
    zeros16 = jnp.zeros((L,), jnp.float32)

    BUF = ((src0, dst0, exv0, rows0, gs0, sr0, sd0, is0, id0),
           (src1, dst1, exv1, rows1, gs1, sr1, sd1, is1, id1),
           (src2, dst2, exv2, rows2, gs2, sr2, sd2, is2, id2))

    def _zero_row(i, carry):
        for c in range(H // L):
            frows0[i, pl.ds(c * L, L)] = zeros16
        return carry
    lax.fori_loop(0, C, _zero_row, 0)
    for k in range(C // L):
        exv0[pl.ds(k * L, L)] = zeros16

    row0 = sid * RPT

    def _zero_rows(nrows):
        full = nrows // C
        for k in range(full):
            pltpu.sync_copy(frows0.at[pl.ds(0, C)],
                            acc_s.at[pl.ds(row0 + C * k, C)])
            pltpu.sync_copy(exv0, den_s.at[pl.ds(row0 + C * k, C)])
        rem = nrows - full * C
        if rem:
            pltpu.sync_copy(frows0.at[pl.ds(0, rem)],
                            acc_s.at[pl.ds(row0 + full * C, rem)])
            pltpu.sync_copy(exv0.at[pl.ds(0, rem)],
                            den_s.at[pl.ds(row0 + full * C, rem)])

    pl.when(sid < NS - 1)(lambda: _zero_rows(RPT))
    pl.when(sid == NS - 1)(lambda: _zero_rows(RPT_LAST))

    pltpu.sync_copy(tab_hbm, tab_l)

    plsc.subcore_barrier()

    def _load_src(b, g):
        pltpu.sync_copy(ei_hbm.at[0, pl.ds(wid * EPW + g * C, C)], BUF[b][0])

    def _load_dst(b, g):
        pltpu.sync_copy(ei_hbm.at[1, pl.ds(wid * EPW + g * C, C)], BUF[b][1])

    def _fire_load_src(b, g):
        pltpu.async_copy(ei_hbm.at[0, pl.ds(wid * EPW + g * C, C)], BUF[b][0],
                         BUF[b][7])

    def _wait_load_src(b):
        pltpu.make_async_copy(ei_hbm.at[0, pl.ds(0, C)], BUF[b][0],
                              BUF[b][7]).wait()

    def _fire_load_dst(b, g):
        pltpu.async_copy(ei_hbm.at[1, pl.ds(wid * EPW + g * C, C)], BUF[b][1],
                         BUF[b][8])

    def _wait_load_dst(b):
        pltpu.make_async_copy(ei_hbm.at[1, pl.ds(0, C)], BUF[b][1],
                              BUF[b][8]).wait()

    def _fire_gather(b):
        pltpu.async_copy(h_hbm.at[BUF[b][0]], BUF[b][3], BUF[b][4])

    def _wait_gather(b):
        pltpu.make_async_copy(h_hbm.at[BUF[b][0]], BUF[b][3], BUF[b][4]).wait()

    def _compute_ex(b):
        buf = BUF[b]
        for gg in range(C // L):
            sidx = buf[0][pl.ds(gg * L, L)]
            didx = buf[1][pl.ds(gg * L, L)]
            ga = plsc.unpack(plsc.bitcast(plsc.load_gather(tab_l, [sidx]),
                                          jnp.bfloat16),
                             format=plsc.PackFormat.INTERLEAVED)
            gd = plsc.unpack(plsc.bitcast(plsc.load_gather(tab_l, [didx]),
                                          jnp.bfloat16),
                             format=plsc.PackFormat.INTERLEAVED)
            s = ga[0].astype(jnp.float32) + gd[1].astype(jnp.float32)
            e = jnp.maximum(s, 0.2 * s)
            buf[2][pl.ds(gg * L, L)] = jnp.exp(e)

    FR = (frows0, frows1)

    def _scale(b, fb):
        rows, exv, frows = BUF[b][3], BUF[b][2], FR[fb]

        @plsc.parallel_loop(0, C, 1, unroll=16)
        def _srow(r):
            spl = plsc.load_gather(exv, [jnp.full((L,), 0, jnp.int32) + r])
            for c in range(H // (2 * L)):
                v = rows[r, pl.ds(c * L, L)]
                lo = jax.lax.bitcast_convert_type(v << 16, jnp.float32)
                hi = jax.lax.bitcast_convert_type(
                    v & jnp.int32(-65536), jnp.float32)
                frows[r, pl.ds(c * L, L)] = lo * spl
                frows[r, pl.ds((H // 2) + c * L, L)] = hi * spl

    def _fire_scatters(b, fb):
        buf = BUF[b]
        pltpu.async_copy(FR[fb], acc_s.at[buf[1]], buf[5], add=True)
        pltpu.async_copy(buf[2], den_s.at[buf[1]], buf[6], add=True)

    def _wait_scatters(b, fb):
        buf = BUF[b]
        pltpu.make_async_copy(FR[fb], acc_s.at[buf[1]], buf[5]).wait()
        pltpu.make_async_copy(buf[2], den_s.at[buf[1]], buf[6]).wait()

    for g0 in range(2):
        _load_src(g0, g0)
        _load_dst(g0, g0)
        _fire_gather(g0)

    def _iter(a, z, fa, g):
        pl.when(g + 2 < NCH)(lambda: _fire_load_src(z, g + 2))
        pl.when(g >= 2)(lambda: _wait_load_dst(a))
        _compute_ex(a)
        _wait_gather(a)
        _scale(a, fa)
        _fire_scatters(a, fa)
        pl.when(g >= 1)(lambda: _wait_scatters(z, 1 - fa))

        def _next_gather():
            _fire_load_dst(z, g + 2)
            _wait_load_src(z)
            _fire_gather(z)
        pl.when(g + 2 < NCH)(_next_gather)

    def _chunk(g, carry):
        for b in range(3):
            for fb in range(2):
                pl.when(jnp.logical_and(g % 3 == b, g % 2 == fb))(
                    lambda b=b, fb=fb: _iter(b, (b + 2) % 3, fb, g))
        return carry
    lax.fori_loop(0, NCH, _chunk, 0)

    _wait_scatters((NCH - 1) % 3, (NCH - 1) % 2)

    plsc.subcore_barrier()

    def _writeback(nrows):
        pltpu.sync_copy(acc_s.at[pl.ds(row0, nrows)],
                        acc_hbm.at[cid, pl.ds(row0, nrows)])
        pltpu.sync_copy(den_s.at[pl.ds(row0, nrows)],
                        den_hbm.at[cid, pl.ds(row0, nrows)])

    pl.when(sid < NS - 1)(lambda: _writeback(RPT))
    pl.when(sid == NS - 1)(lambda: _writeback(RPT_LAST))


def _edge_phase(h, tab, edge_index):
    f32 = jnp.float32
    i32 = jnp.int32
    call = pl.kernel(
        _edge_body,
        out_type=(
            jax.ShapeDtypeStruct((NC, N, H), f32),
            jax.ShapeDtypeStruct((NC, N), f32),
        ),
        mesh=plsc.VectorSubcoreMesh(core_axis_name="c", subcore_axis_name="s"),
        compiler_params=pltpu.CompilerParams(needs_layout_passes=False,
                                             use_tc_tiling_on_sc=False),
        scratch_types=[
            pltpu.VMEM((N,), i32),
            pltpu.VMEM((C,), i32),
            pltpu.VMEM((C,), i32),
            pltpu.VMEM((C,), i32),
            pltpu.VMEM((C,), i32),
            pltpu.VMEM((C,), i32),
            pltpu.VMEM((C,), i32),
            pltpu.VMEM((C,), f32),
            pltpu.VMEM((C,), f32),
            pltpu.VMEM((C,), f32),
            pltpu.VMEM((C, H // 2), i32),
            pltpu.VMEM((C, H // 2), i32),
            pltpu.VMEM((C, H // 2), i32),
            pltpu.VMEM((C, H), f32),
            pltpu.VMEM((C, H), f32),
            pltpu.VMEM_SHARED((N, H), f32),
            pltpu.VMEM_SHARED((N,), f32),
            pltpu.SemaphoreType.DMA,
            pltpu.SemaphoreType.DMA,
            pltpu.SemaphoreType.DMA,
            pltpu.SemaphoreType.DMA,
            pltpu.SemaphoreType.DMA,
            pltpu.SemaphoreType.DMA,
            pltpu.SemaphoreType.DMA,
            pltpu.SemaphoreType.DMA,
            pltpu.SemaphoreType.DMA,
            pltpu.SemaphoreType.DMA,
            pltpu.SemaphoreType.DMA,
            pltpu.SemaphoreType.DMA,
            pltpu.SemaphoreType.DMA,
            pltpu.SemaphoreType.DMA,
            pltpu.SemaphoreType.DMA,
        ],
    )
    return call(h, tab, edge_index)


def _pack_h_tc(h):
    b = jax.lax.bitcast_convert_type(h, jnp.int32)
    r = (b + 0x7FFF + ((b >> 16) & 1)) >> 16
    lo = r[:, :H // 2] & 0xFFFF
    hi = r[:, H // 2:] << 16
    return hi | lo


def _pack_logits_tc(als, ald):
    ab = jax.lax.bitcast_convert_type(als, jnp.int32)
    db = jax.lax.bitcast_convert_type(ald, jnp.int32)
    ar = (ab + 0x7FFF + ((ab >> 16) & 1)) >> 16
    dr = (db + 0x7FFF + ((db >> 16) & 1)) >> 16
    return (dr << 16) | (ar & 0xFFFF)


def _prologue_body(xd_ref, xt_ref, ws_ref, wd_ref, asrc_ref, adst_ref,
                   h_ref, tab_ref):
    h_src = jnp.dot(xd_ref[...], ws_ref[...], preferred_element_type=jnp.float32)
    h_ref[...] = _pack_h_tc(h_src)
    als = jnp.dot(h_src, asrc_ref[...], preferred_element_type=jnp.float32)
    h_dst = jnp.dot(xt_ref[...], wd_ref[...], preferred_element_type=jnp.float32)
    ald = jnp.dot(h_dst, adst_ref[...], preferred_element_type=jnp.float32)
    tab_ref[...] = _pack_logits_tc(als, ald)


def _prologue(x_data, x_tasks, Wd_src, Wd_dst, ad_src, ad_dst):
    return pl.pallas_call(
        _prologue_body,
        out_shape=(
            jax.ShapeDtypeStruct((N, H // 2), jnp.int32),
            jax.ShapeDtypeStruct((N,), jnp.int32),
        ),
    )(x_data, x_tasks, Wd_src, Wd_dst, ad_src, ad_dst)


def _finalize(acc_ref, den_ref):
    t = (acc_ref[0] + acc_ref[1]) / (den_ref[0] + den_ref[1] + 1e-16)[:, None]
    return jnp.where(t > 0, t, jnp.exp(t) - 1.0)


def _mid_body(acc_ref, den_ref, w_ref, asrc_ref, adst_ref,
              h_ref, tab_ref):
    t = _finalize(acc_ref, den_ref)
    h = jnp.dot(t, w_ref[...], preferred_element_type=jnp.float32)
    h_ref[...] = _pack_h_tc(h)
    als = jnp.dot(h, asrc_ref[...], preferred_element_type=jnp.float32)
    ald = jnp.dot(h, adst_ref[...], preferred_element_type=jnp.float32)
    tab_ref[...] = _pack_logits_tc(als, ald)


def _mid(acc, den, W, a_src, a_dst):
    return pl.pallas_call(
        _mid_body,
        out_shape=(
            jax.ShapeDtypeStruct((N, H // 2), jnp.int32),
            jax.ShapeDtypeStruct((N,), jnp.int32),
        ),
    )(acc, den, W, a_src, a_dst)


def _epilogue_body(acc_ref, den_ref, xdev_ref, wdev_ref, counts_ref, out_ref):
    t = _finalize(acc_ref, den_ref)
    out_ref[0:H] = t[0]
    cnt = jnp.maximum(counts_ref[0, 0], 1.0)
    out_ref[H:2 * H] = jnp.sum(t, axis=0) / cnt
    dev = jnp.maximum(
        jnp.dot(xdev_ref[...], wdev_ref[...], preferred_element_type=jnp.float32), 0.0)
    out_ref[2 * H:] = dev.reshape(-1)


def _epilogue(acc, den, x_devices, W_dev, counts):
    ndev = x_devices.shape[0]
    return pl.pallas_call(
        _epilogue_body,
        out_shape=jax.ShapeDtypeStruct((2 * H + ndev * H,), jnp.float32),
    )(acc, den, x_devices, W_dev, counts)


def kernel(x_data, x_tasks, x_devices, counts, edge_index_dt, edge_index_tt,
           Wd_src, Wd_dst, ad_src, ad_dst,
           W1, a1_src, a1_dst,
           W2, a2_src, a2_dst,
           W_dev):
    ei_dt = edge_index_dt.astype(jnp.int32)
    ei_tt = edge_index_tt.astype(jnp.int32)

    h, tab = _prologue(x_data, x_tasks, Wd_src, Wd_dst, ad_src, ad_dst)
    acc, den = _edge_phase(h, tab, ei_dt)
    h, tab = _mid(acc, den, W1, a1_src, a1_dst)
    acc, den = _edge_phase(h, tab, ei_tt)
    h, tab = _mid(acc, den, W2, a2_src, a2_dst)
    acc, den = _edge_phase(h, tab, ei_tt)
    return _epilogue(acc, den, x_devices, W_dev, counts)
